# Initial kernel scaffold; baseline (speedup 1.0000x reference)
import jax
import jax.numpy as jnp
from jax.experimental import pallas as pl


def kernel(idx, read_depth, emb, W1, b1, bn1_g, bn1_b, bn1_m, bn1_v, W2, b2, bn2_g, bn2_b, bn2_m, bn2_v):
    raise NotImplementedError("write your pallas kernel here")


# speedup vs baseline: 11.4050x; 11.4050x over previous
import functools

import jax
import jax.numpy as jnp
from jax import lax
from jax.experimental import pallas as pl
from jax.experimental.pallas import tpu as pltpu
from jax.experimental.pallas import tpu_sc as plsc

_EPS = 1e-5

_NUM_SC = 2
_NUM_SUBCORES = 16
_LANES = 16


@functools.lru_cache(maxsize=None)
def _make_pool(B, H, D):
    NW = _NUM_SC * _NUM_SUBCORES
    BPW = B // NW
    assert B % NW == 0 and BPW % 2 == 0 and D % _LANES == 0
    C0 = min(128, H)
    C1 = H - C0
    NCH = D // _LANES
    mesh = plsc.VectorSubcoreMesh(core_axis_name="c", subcore_axis_name="s")

    @functools.partial(
        pl.kernel,
        out_type=jax.ShapeDtypeStruct((B, D), jnp.float32),
        mesh=mesh,
        scratch_types=[
            pltpu.VMEM((BPW, H), jnp.int32),
            pltpu.VMEM((2, H, D), jnp.float32),
            pltpu.VMEM((BPW, D), jnp.float32),
            pltpu.SemaphoreType.DMA,
            pltpu.SemaphoreType.DMA,
            pltpu.SemaphoreType.DMA,
        ],
        compiler_params=pltpu.CompilerParams(use_tc_tiling_on_sc=False),
    )
    def pool(idx_hbm, emb_hbm, out_hbm, idx_v, rows_v, out_v, sem_i, sem0, sem1):
        wid = lax.axis_index("s") * _NUM_SC + lax.axis_index("c")
        base = wid * BPW
        pltpu.async_copy(idx_hbm.at[pl.ds(base, BPW)], idx_v, sem_i).wait()
        sems = (sem0, sem1)

        def fire(b, p):
            pltpu.make_async_copy(
                emb_hbm.at[idx_v.at[b, pl.ds(0, C0)]],
                rows_v.at[p, pl.ds(0, C0)], sems[p]).start()
            if C1:
                pltpu.make_async_copy(
                    emb_hbm.at[idx_v.at[b, pl.ds(C0, C1)]],
                    rows_v.at[p, pl.ds(C0, C1)], sems[p]).start()

        def wait(p):
            pltpu.make_async_copy(
                emb_hbm.at[idx_v.at[0, pl.ds(0, C0)]],
                rows_v.at[p, pl.ds(0, C0)], sems[p]).wait()
            if C1:
                pltpu.make_async_copy(
                    emb_hbm.at[idx_v.at[0, pl.ds(C0, C1)]],
                    rows_v.at[p, pl.ds(C0, C1)], sems[p]).wait()

        def reduce(b, p):
            def body(i, accs):
                return tuple(
                    a + rows_v[p, i, pl.ds(_LANES * c, _LANES)]
                    for c, a in enumerate(accs)
                )
            init = tuple(jnp.zeros((_LANES,), jnp.float32) for _ in range(NCH))
            accs = lax.fori_loop(0, H, body, init, unroll=4)
            for c in range(NCH):
                out_v[b, pl.ds(_LANES * c, _LANES)] = accs[c]

        fire(0, 0)

        @pl.loop(0, BPW, step=2)
        def _(b):
            wait(0)
            fire(b + 1, 1)
            reduce(b, 0)
            wait(1)

            @pl.when(b + 2 < BPW)
            def _():
                fire(b + 2, 0)

            reduce(b + 1, 1)

        pltpu.sync_copy(out_v, out_hbm.at[pl.ds(base, BPW)])

    return pool


def _mlp_body(idx_ref, pooled_ref, w1_ref, w1r_ref, c1_ref, w2_ref, c2_ref,
              loc_ref, scale_ref):
    D = pooled_ref.shape[1]
    cnt = jnp.sum((idx_ref[...] > 0).astype(jnp.float32), axis=1, keepdims=True)
    x = pooled_ref[...] / cnt
    h = jnp.dot(x, w1_ref[...], preferred_element_type=jnp.float32)
    h = h + jnp.log(cnt) * w1r_ref[...] + c1_ref[...]
    h = jnp.maximum(h, 0.0)
    o = jnp.dot(h, w2_ref[...], preferred_element_type=jnp.float32) + c2_ref[...]
    loc_ref[...] = o[:, :D]
    scale_ref[...] = jax.nn.softplus(o[:, D:])


def kernel(idx, read_depth, emb, W1, b1, bn1_g, bn1_b, bn1_m, bn1_v,
           W2, b2, bn2_g, bn2_b, bn2_m, bn2_v):
    del read_depth
    idx = idx.astype(jnp.int32)
    B, H = idx.shape
    D = emb.shape[1]
    HID = W1.shape[1]

    a1 = bn1_g / jnp.sqrt(bn1_v + _EPS)
    W1s = W1 * a1[None, :]
    c1 = ((b1 - bn1_m) * a1 + bn1_b)[None, :]
    a2 = bn2_g / jnp.sqrt(bn2_v + _EPS)
    W2s = W2 * a2[None, :]
    c2 = ((b2 - bn2_m) * a2 + bn2_b)[None, :]

    pooled = _make_pool(B, H, D)(idx, emb)

    BB = 512
    grid = (B // BB,)
    loc, scale = pl.pallas_call(
        _mlp_body,
        grid=grid,
        in_specs=[
            pl.BlockSpec((BB, H), lambda i: (i, 0)),
            pl.BlockSpec((BB, D), lambda i: (i, 0)),
            pl.BlockSpec((D, HID), lambda i: (0, 0)),
            pl.BlockSpec((1, HID), lambda i: (0, 0)),
            pl.BlockSpec((1, HID), lambda i: (0, 0)),
            pl.BlockSpec((HID, 2 * D), lambda i: (0, 0)),
            pl.BlockSpec((1, 2 * D), lambda i: (0, 0)),
        ],
        out_specs=[
            pl.BlockSpec((BB, D), lambda i: (i, 0)),
            pl.BlockSpec((BB, D), lambda i: (i, 0)),
        ],
        out_shape=[
            jax.ShapeDtypeStruct((B, D), jnp.float32),
            jax.ShapeDtypeStruct((B, D), jnp.float32),
        ],
    )(idx, pooled, W1s[:D], W1s[D:D + 1], c1, W2s, c2)
    return loc, scale


# speedup vs baseline: 11.4188x; 1.0012x over previous
import functools

import jax
import jax.numpy as jnp
from jax import lax
from jax.experimental import pallas as pl
from jax.experimental.pallas import tpu as pltpu
from jax.experimental.pallas import tpu_sc as plsc

_EPS = 1e-5

_NUM_SC = 2
_NUM_SUBCORES = 16
_LANES = 16


@functools.lru_cache(maxsize=None)
def _make_pool(B, H, D):
    NW = _NUM_SC * _NUM_SUBCORES
    BPW = B // NW
    assert B % NW == 0 and BPW % 2 == 0 and D % _LANES == 0
    C0 = min(104, H)
    C1 = H - C0
    NCH = D // _LANES
    mesh = plsc.VectorSubcoreMesh(core_axis_name="c", subcore_axis_name="s")

    @functools.partial(
        pl.kernel,
        out_type=jax.ShapeDtypeStruct((B, D), jnp.float32),
        mesh=mesh,
        scratch_types=[
            pltpu.VMEM((BPW, H), jnp.int32),
            pltpu.VMEM((2, H, D), jnp.float32),
            pltpu.VMEM((BPW, D), jnp.float32),
            pltpu.SemaphoreType.DMA,
            pltpu.SemaphoreType.DMA,
            pltpu.SemaphoreType.DMA,
        ],
        compiler_params=pltpu.CompilerParams(use_tc_tiling_on_sc=False),
    )
    def pool(idx_hbm, emb_hbm, out_hbm, idx_v, rows_v, out_v, sem_i, sem0, sem1):
        wid = lax.axis_index("s") * _NUM_SC + lax.axis_index("c")
        base = wid * BPW
        pltpu.async_copy(idx_hbm.at[pl.ds(base, BPW)], idx_v, sem_i).wait()
        sems = (sem0, sem1)

        def fire(b, p):
            pltpu.make_async_copy(
                emb_hbm.at[idx_v.at[b, pl.ds(0, C0)]],
                rows_v.at[p, pl.ds(0, C0)], sems[p]).start()
            if C1:
                pltpu.make_async_copy(
                    emb_hbm.at[idx_v.at[b, pl.ds(C0, C1)]],
                    rows_v.at[p, pl.ds(C0, C1)], sems[p]).start()

        def wait(p):
            pltpu.make_async_copy(
                emb_hbm.at[idx_v.at[0, pl.ds(0, C0)]],
                rows_v.at[p, pl.ds(0, C0)], sems[p]).wait()
            if C1:
                pltpu.make_async_copy(
                    emb_hbm.at[idx_v.at[0, pl.ds(C0, C1)]],
                    rows_v.at[p, pl.ds(C0, C1)], sems[p]).wait()

        def reduce(b, p):
            def body(i, accs):
                return tuple(
                    a + rows_v[p, i, pl.ds(_LANES * c, _LANES)]
                    for c, a in enumerate(accs)
                )
            init = tuple(jnp.zeros((_LANES,), jnp.float32) for _ in range(NCH))
            accs = lax.fori_loop(0, H, body, init, unroll=8)
            for c in range(NCH):
                out_v[b, pl.ds(_LANES * c, _LANES)] = accs[c]

        fire(0, 0)

        @pl.loop(0, BPW, step=2)
        def _(b):
            wait(0)
            fire(b + 1, 1)
            reduce(b, 0)
            wait(1)

            @pl.when(b + 2 < BPW)
            def _():
                fire(b + 2, 0)

            reduce(b + 1, 1)

        pltpu.sync_copy(out_v, out_hbm.at[pl.ds(base, BPW)])

    return pool


def _mlp_body(idx_ref, pooled_ref, w1_ref, w1r_ref, c1_ref, w2_ref, c2_ref,
              loc_ref, scale_ref):
    D = pooled_ref.shape[1]
    cnt = jnp.sum((idx_ref[...] > 0).astype(jnp.float32), axis=1, keepdims=True)
    x = pooled_ref[...] / cnt
    h = jnp.dot(x, w1_ref[...], preferred_element_type=jnp.float32)
    h = h + jnp.log(cnt) * w1r_ref[...] + c1_ref[...]
    h = jnp.maximum(h, 0.0)
    o = jnp.dot(h, w2_ref[...], preferred_element_type=jnp.float32) + c2_ref[...]
    loc_ref[...] = o[:, :D]
    scale_ref[...] = jax.nn.softplus(o[:, D:])


def kernel(idx, read_depth, emb, W1, b1, bn1_g, bn1_b, bn1_m, bn1_v,
           W2, b2, bn2_g, bn2_b, bn2_m, bn2_v):
    del read_depth
    idx = idx.astype(jnp.int32)
    B, H = idx.shape
    D = emb.shape[1]
    HID = W1.shape[1]

    a1 = bn1_g / jnp.sqrt(bn1_v + _EPS)
    W1s = W1 * a1[None, :]
    c1 = ((b1 - bn1_m) * a1 + bn1_b)[None, :]
    a2 = bn2_g / jnp.sqrt(bn2_v + _EPS)
    W2s = W2 * a2[None, :]
    c2 = ((b2 - bn2_m) * a2 + bn2_b)[None, :]

    pooled = _make_pool(B, H, D)(idx, emb)

    BB = 512
    grid = (B // BB,)
    loc, scale = pl.pallas_call(
        _mlp_body,
        grid=grid,
        in_specs=[
            pl.BlockSpec((BB, H), lambda i: (i, 0)),
            pl.BlockSpec((BB, D), lambda i: (i, 0)),
            pl.BlockSpec((D, HID), lambda i: (0, 0)),
            pl.BlockSpec((1, HID), lambda i: (0, 0)),
            pl.BlockSpec((1, HID), lambda i: (0, 0)),
            pl.BlockSpec((HID, 2 * D), lambda i: (0, 0)),
            pl.BlockSpec((1, 2 * D), lambda i: (0, 0)),
        ],
        out_specs=[
            pl.BlockSpec((BB, D), lambda i: (i, 0)),
            pl.BlockSpec((BB, D), lambda i: (i, 0)),
        ],
        out_shape=[
            jax.ShapeDtypeStruct((B, D), jnp.float32),
            jax.ShapeDtypeStruct((B, D), jnp.float32),
        ],
    )(idx, pooled, W1s[:D], W1s[D:D + 1], c1, W2s, c2)
    return loc, scale


# speedup vs baseline: 16.8267x; 1.4736x over previous
import functools

import jax
import jax.numpy as jnp
from jax import lax
from jax.experimental import pallas as pl
from jax.experimental.pallas import tpu as pltpu
from jax.experimental.pallas import tpu_sc as plsc

_EPS = 1e-5

_NUM_SC = 2
_NUM_SUBCORES = 16
_LANES = 16


@functools.lru_cache(maxsize=None)
def _make_pool(B, H, D):
    NW = _NUM_SC * _NUM_SUBCORES
    BPW = B // NW
    assert B % NW == 0 and BPW % 2 == 0 and D % _LANES == 0
    C0 = min(104, H)
    C1 = H - C0
    NCH = D // _LANES
    mesh = plsc.VectorSubcoreMesh(core_axis_name="c", subcore_axis_name="s")

    @functools.partial(
        pl.kernel,
        out_type=jax.ShapeDtypeStruct((B, D), jnp.float32),
        mesh=mesh,
        scratch_types=[
            pltpu.VMEM((BPW, H), jnp.int32),
            pltpu.VMEM((4, H, D), jnp.float32),
            pltpu.VMEM((BPW, D), jnp.float32),
            pltpu.SemaphoreType.DMA,
            pltpu.SemaphoreType.DMA,
            pltpu.SemaphoreType.DMA,
            pltpu.SemaphoreType.DMA,
            pltpu.SemaphoreType.DMA,
        ],
        compiler_params=pltpu.CompilerParams(use_tc_tiling_on_sc=False),
    )
    def pool(idx_hbm, emb_hbm, out_hbm, idx_v, rows_v, out_v, sem_i,
             sem0, sem1, sem2, sem3):
        wid = lax.axis_index("s") * _NUM_SC + lax.axis_index("c")
        base = wid * BPW
        pltpu.async_copy(idx_hbm.at[pl.ds(base, BPW)], idx_v, sem_i).wait()
        sems = (sem0, sem1, sem2, sem3)

        def fire(b, p):
            pltpu.make_async_copy(
                emb_hbm.at[idx_v.at[b, pl.ds(0, C0)]],
                rows_v.at[p, pl.ds(0, C0)], sems[p]).start()
            if C1:
                pltpu.make_async_copy(
                    emb_hbm.at[idx_v.at[b, pl.ds(C0, C1)]],
                    rows_v.at[p, pl.ds(C0, C1)], sems[p]).start()

        def wait(p):
            pltpu.make_async_copy(
                emb_hbm.at[idx_v.at[0, pl.ds(0, C0)]],
                rows_v.at[p, pl.ds(0, C0)], sems[p]).wait()
            if C1:
                pltpu.make_async_copy(
                    emb_hbm.at[idx_v.at[0, pl.ds(C0, C1)]],
                    rows_v.at[p, pl.ds(C0, C1)], sems[p]).wait()

        def reduce(b, p):
            def body(i, accs):
                return tuple(
                    a + rows_v[p, i, pl.ds(_LANES * c, _LANES)]
                    for c, a in enumerate(accs)
                )
            init = tuple(jnp.zeros((_LANES,), jnp.float32) for _ in range(NCH))
            accs = lax.fori_loop(0, H, body, init, unroll=8)
            for c in range(NCH):
                out_v[b, pl.ds(_LANES * c, _LANES)] = accs[c]

        for p in range(4):
            fire(p, p)

        @pl.loop(0, BPW, step=4)
        def _(b):
            for p in range(4):
                wait(p)
                reduce(b + p, p)

                @pl.when(b + p + 4 < BPW)
                def _():
                    fire(b + p + 4, p)

        pltpu.sync_copy(out_v, out_hbm.at[pl.ds(base, BPW)])

    return pool


def _mlp_body(idx_ref, pooled_ref, w1_ref, w1r_ref, c1_ref, w2_ref, c2_ref,
              loc_ref, scale_ref):
    D = pooled_ref.shape[1]
    cnt = jnp.sum((idx_ref[...] > 0).astype(jnp.float32), axis=1, keepdims=True)
    x = pooled_ref[...] / cnt
    h = jnp.dot(x, w1_ref[...], preferred_element_type=jnp.float32)
    h = h + jnp.log(cnt) * w1r_ref[...] + c1_ref[...]
    h = jnp.maximum(h, 0.0)
    o = jnp.dot(h, w2_ref[...], preferred_element_type=jnp.float32) + c2_ref[...]
    loc_ref[...] = o[:, :D]
    scale_ref[...] = jax.nn.softplus(o[:, D:])


def kernel(idx, read_depth, emb, W1, b1, bn1_g, bn1_b, bn1_m, bn1_v,
           W2, b2, bn2_g, bn2_b, bn2_m, bn2_v):
    del read_depth
    idx = idx.astype(jnp.int32)
    B, H = idx.shape
    D = emb.shape[1]
    HID = W1.shape[1]

    a1 = bn1_g / jnp.sqrt(bn1_v + _EPS)
    W1s = W1 * a1[None, :]
    c1 = ((b1 - bn1_m) * a1 + bn1_b)[None, :]
    a2 = bn2_g / jnp.sqrt(bn2_v + _EPS)
    W2s = W2 * a2[None, :]
    c2 = ((b2 - bn2_m) * a2 + bn2_b)[None, :]

    pooled = _make_pool(B, H, D)(idx, emb)

    BB = 512
    grid = (B // BB,)
    loc, scale = pl.pallas_call(
        _mlp_body,
        grid=grid,
        in_specs=[
            pl.BlockSpec((BB, H), lambda i: (i, 0)),
            pl.BlockSpec((BB, D), lambda i: (i, 0)),
            pl.BlockSpec((D, HID), lambda i: (0, 0)),
            pl.BlockSpec((1, HID), lambda i: (0, 0)),
            pl.BlockSpec((1, HID), lambda i: (0, 0)),
            pl.BlockSpec((HID, 2 * D), lambda i: (0, 0)),
            pl.BlockSpec((1, 2 * D), lambda i: (0, 0)),
        ],
        out_specs=[
            pl.BlockSpec((BB, D), lambda i: (i, 0)),
            pl.BlockSpec((BB, D), lambda i: (i, 0)),
        ],
        out_shape=[
            jax.ShapeDtypeStruct((B, D), jnp.float32),
            jax.ShapeDtypeStruct((B, D), jnp.float32),
        ],
    )(idx, pooled, W1s[:D], W1s[D:D + 1], c1, W2s, c2)
    return loc, scale
